# baseline (speedup 1.0000x reference)
import functools

import jax
import jax.numpy as jnp
from jax import lax
from jax.experimental import pallas as pl
from jax.experimental.pallas import tpu as pltpu
from jax.experimental.pallas import tpu_sc as plsc

NC = 2
NS = 16
L = 16
NW = NC * NS

EMBED_DIM = 32
MARGIN = 1.0


def _rsqrt(x):
    i = plsc.bitcast(x, jnp.int32)
    i = jnp.int32(0x5F3759DF) - (i >> 1)
    y = plsc.bitcast(i, jnp.float32)
    for _ in range(3):
        y = y * (jnp.float32(1.5) - (jnp.float32(0.5) * x * y) * y)
    return y


def _make_sc_kernel(batch):
    assert batch % (8 * NW) == 0
    bpw = batch // NW
    groups = bpw // L

    mesh = plsc.VectorSubcoreMesh(
        core_axis_name="c", subcore_axis_name="s", num_cores=NC, num_subcores=NS
    )

    @functools.partial(
        pl.kernel,
        out_type=jax.ShapeDtypeStruct((1, batch), jnp.float32),
        mesh=mesh,
        scratch_types=[
            pltpu.VMEM((bpw,), jnp.int32),
            pltpu.VMEM((bpw,), jnp.int32),
            pltpu.VMEM((bpw,), jnp.int32),
            pltpu.VMEM((bpw,), jnp.int32),
            pltpu.VMEM((bpw,), jnp.int32),
            pltpu.VMEM((bpw,), jnp.int32),
            pltpu.VMEM((bpw, EMBED_DIM), jnp.float32),
            pltpu.VMEM((bpw, EMBED_DIM), jnp.float32),
            pltpu.VMEM((bpw, EMBED_DIM), jnp.float32),
            pltpu.VMEM((bpw, EMBED_DIM), jnp.float32),
            pltpu.VMEM((bpw, EMBED_DIM), jnp.float32),
            pltpu.VMEM((bpw, EMBED_DIM), jnp.float32),
            pltpu.VMEM((bpw,), jnp.float32),
            pltpu.SemaphoreType.DMA,
        ],
        compiler_params=pltpu.CompilerParams(
            needs_layout_passes=False, use_tc_tiling_on_sc=False
        ),
    )
    def sc_kernel(
        hp_hbm,
        lp_hbm,
        tp_hbm,
        hn_hbm,
        ln_hbm,
        tn_hbm,
        ent_hbm,
        lab_hbm,
        out_hbm,
        hp_i,
        lp_i,
        tp_i,
        hn_i,
        ln_i,
        tn_i,
        hp_v,
        lp_v,
        tp_v,
        hn_v,
        ln_v,
        tn_v,
        out_v,
        sem,
    ):
        wid = lax.axis_index("s") * NC + lax.axis_index("c")
        base = wid * bpw

        pltpu.sync_copy(hp_hbm.at[pl.ds(base, bpw)], hp_i)
        pltpu.sync_copy(lp_hbm.at[pl.ds(base, bpw)], lp_i)
        pltpu.sync_copy(tp_hbm.at[pl.ds(base, bpw)], tp_i)
        pltpu.sync_copy(hn_hbm.at[pl.ds(base, bpw)], hn_i)
        pltpu.sync_copy(ln_hbm.at[pl.ds(base, bpw)], ln_i)
        pltpu.sync_copy(tn_hbm.at[pl.ds(base, bpw)], tn_i)

        cps = [
            pltpu.async_copy(ent_hbm.at[hp_i], hp_v, sem),
            pltpu.async_copy(lab_hbm.at[lp_i], lp_v, sem),
            pltpu.async_copy(ent_hbm.at[tp_i], tp_v, sem),
            pltpu.async_copy(ent_hbm.at[hn_i], hn_v, sem),
            pltpu.async_copy(lab_hbm.at[ln_i], ln_v, sem),
            pltpu.async_copy(ent_hbm.at[tn_i], tn_v, sem),
        ]
        for cp in cps:
            cp.wait()

        cols = [jnp.full((L,), j, dtype=jnp.int32) for j in range(EMBED_DIM)]

        def distance(h_v, l_v, t_v, rid):
            z = jnp.zeros((L,), jnp.float32)
            hh = tt = ll = hl = ht = lt = z
            for j in range(EMBED_DIM):
                h = plsc.load_gather(h_v, [rid, cols[j]])
                l = plsc.load_gather(l_v, [rid, cols[j]])
                t = plsc.load_gather(t_v, [rid, cols[j]])
                hh = hh + h * h
                tt = tt + t * t
                ll = ll + l * l
                hl = hl + h * l
                ht = ht + h * t
                lt = lt + l * t
            a = _rsqrt(hh)
            b = _rsqrt(tt)
            two = jnp.float32(2.0)
            dsq = two + ll + two * a * hl - two * (a * b) * ht - two * b * lt
            dsq = jnp.maximum(dsq, jnp.float32(0.0))
            return dsq * _rsqrt(dsq)

        def group(g, carry):
            rid = g * L + lax.iota(jnp.int32, L)
            dp = distance(hp_v, lp_v, tp_v, rid)
            dn = distance(hn_v, ln_v, tn_v, rid)
            loss = jnp.maximum(jnp.float32(MARGIN) + dp - dn, jnp.float32(0.0))
            out_v[pl.ds(g * L, L)] = loss
            return carry

        lax.fori_loop(0, groups, group, 0)

        pltpu.sync_copy(out_v, out_hbm.at[0, pl.ds(base, bpw)])

    return sc_kernel


def kernel(positive, negative, embed_entity, embed_label):
    batch = positive.shape[0]
    sc = _make_sc_kernel(batch)
    return sc(
        positive[:, 0],
        positive[:, 1],
        positive[:, 2],
        negative[:, 0],
        negative[:, 1],
        negative[:, 2],
        embed_entity,
        embed_label,
    )
